# baseline (speedup 1.0000x reference)
import jax
import jax.numpy as jnp
from jax import lax
from jax.experimental import pallas as pl
from jax.experimental.pallas import tpu as pltpu
from jax.experimental.pallas import tpu_sc as plsc

BATCH = 16384
DIM = 64
NC = 2
NS = 16
NW = NC * NS
BPW = BATCH // NW
CW = 128
CH = BPW // CW


def _gather_body(users_hbm, neg_hbm, ut_hbm, et_hbm,
                 u_out, n_out, uidx, nidx, ubuf, nbuf, sem):
    wid = lax.axis_index("s") * NC + lax.axis_index("c")
    base = wid * BPW
    for c in range(CH):
        pltpu.sync_copy(users_hbm.at[pl.ds(base + c * CW, CW)], uidx.at[c])
        pltpu.sync_copy(neg_hbm.at[pl.ds(base + c * CW, CW)], nidx.at[c])
    for c in range(CH):
        pltpu.async_copy(ut_hbm.at[uidx.at[c]],
                         ubuf.at[pl.ds(c * CW, CW)], sem)
        pltpu.async_copy(et_hbm.at[nidx.at[c]],
                         nbuf.at[pl.ds(c * CW, CW)], sem)
    for c in range(CH):
        pltpu.make_async_copy(ut_hbm.at[uidx.at[c]],
                              ubuf.at[pl.ds(c * CW, CW)], sem).wait()
        pltpu.make_async_copy(et_hbm.at[nidx.at[c]],
                              nbuf.at[pl.ds(c * CW, CW)], sem).wait()
    pltpu.sync_copy(ubuf, u_out.at[pl.ds(base, BPW)])
    pltpu.sync_copy(nbuf, n_out.at[pl.ds(base, BPW)])


_gather = pl.kernel(
    _gather_body,
    mesh=plsc.VectorSubcoreMesh(core_axis_name="c", subcore_axis_name="s"),
    out_type=(
        jax.ShapeDtypeStruct((BATCH, DIM), jnp.float32),
        jax.ShapeDtypeStruct((BATCH, DIM), jnp.float32),
    ),
    compiler_params=pltpu.CompilerParams(use_tc_tiling_on_sc=False),
    scratch_types=[
        pltpu.VMEM((CH, CW), jnp.int32),
        pltpu.VMEM((CH, CW), jnp.int32),
        pltpu.VMEM((BPW, DIM), jnp.float32),
        pltpu.VMEM((BPW, DIM), jnp.float32),
        pltpu.SemaphoreType.DMA,
    ],
)


def _loss_body(u_ref, n_ref, out_ref):
    z = u_ref[...] * n_ref[...]
    sp = jnp.maximum(z, 0.0) + jnp.log1p(jnp.exp(-jnp.abs(z)))
    out_ref[0, 0] = jnp.mean(sp)


def kernel(users, pos, neg, user_table, entity_table):
    del pos
    u_g, n_g = _gather(users.astype(jnp.int32), neg.astype(jnp.int32),
                       user_table, entity_table)
    loss = pl.pallas_call(
        _loss_body,
        out_shape=jax.ShapeDtypeStruct((1, 1), jnp.float32),
        out_specs=pl.BlockSpec(memory_space=pltpu.SMEM),
    )(u_g, n_g)
    return loss[0, 0]
